# baseline (speedup 1.0000x reference)
import functools

import jax
import jax.numpy as jnp
from jax import lax
from jax.experimental import pallas as pl
from jax.experimental.pallas import tpu as pltpu
from jax.experimental.pallas import tpu_sc as plsc

NUM_NODES = 1000000
STATE_DIM = 64
BATCH = 16384

_info = plsc.get_sparse_core_info()
_NC, _NS, _L = _info.num_cores, _info.num_subcores, _info.num_lanes
_NW = _NC * _NS
_SC_ROWS = 8704
_K = 16
_NSEM = 16


def _make_sc_gather(sc_rows):
    mesh = plsc.VectorSubcoreMesh(core_axis_name="c", subcore_axis_name="s")
    b_w = sc_rows // _NW
    ng = b_w // _K if b_w else 0
    out_types = [jax.ShapeDtypeStruct((BATCH,), jnp.int32)]
    scratch = [
        pltpu.VMEM((BATCH // _NW,), jnp.int32),
        pltpu.VMEM((BATCH // _NW,), jnp.int32),
        pltpu.SemaphoreType.DMA,
    ]
    if sc_rows:
        out_types.append(
            jax.ShapeDtypeStruct((sc_rows, STATE_DIM), jnp.float32))
        scratch += [
            pltpu.VMEM((b_w, STATE_DIM), jnp.float32),
            pltpu.SemaphoreType.DMA,
            pltpu.SemaphoreType.DMA,
        ]

    @functools.partial(
        pl.kernel,
        mesh=mesh,
        out_type=tuple(out_types),
        scratch_types=scratch,
        compiler_params=pltpu.CompilerParams(needs_layout_passes=False),
    )
    def k(mem_hbm, lu_hbm, nid_hbm, lu_out, *rest):
        wid = lax.axis_index("s") * _NC + lax.axis_index("c")
        lu_b = BATCH // _NW
        lu_base = wid * lu_b
        if sc_rows:
            mem_out, idx_v, lu_v, s_lu, rows_v, s_a, s_b = rest
        else:
            idx_v, lu_v, s_lu = rest
        pltpu.sync_copy(nid_hbm.at[pl.ds(lu_base, lu_b)], idx_v)
        lu_cp = pltpu.async_copy(lu_hbm.at[idx_v], lu_v, s_lu)

        if sc_rows:
            base = wid * b_w

            def body(g, _):
                vec = idx_v[pl.ds(g * _K, _K)]
                for b in range(_K):
                    v = vec[b]
                    pltpu.async_copy(
                        mem_hbm.at[v], rows_v.at[g * _K + b], s_a)
                return _

            lax.fori_loop(0, ng, body, 0)
            pltpu.make_async_copy(
                mem_hbm.at[pl.ds(0, b_w)], rows_v, s_a).wait()
            pltpu.sync_copy(rows_v, mem_out.at[pl.ds(base, b_w)])

        lu_cp.wait()
        pltpu.sync_copy(lu_v, lu_out.at[pl.ds(lu_base, lu_b)])

    return k


_TC_UNROLL = 8


def _make_tc_gather(n_rows):
    def body(nid_smem, mem_any, out_any, rows_v, sem, sem_out):
        def loop(jo, carry):
            for b in range(_TC_UNROLL):
                j = jo * _TC_UNROLL + b
                v = nid_smem[j]
                pltpu.make_async_copy(
                    mem_any.at[pl.ds(v, 1)], rows_v.at[pl.ds(j, 1)],
                    sem).start()
            return carry

        lax.fori_loop(0, n_rows // _TC_UNROLL, loop, 0)
        pltpu.make_async_copy(
            mem_any.at[pl.ds(0, n_rows)], rows_v, sem).wait()
        out_cp = pltpu.make_async_copy(rows_v, out_any, sem_out)
        out_cp.start()
        out_cp.wait()

    return pl.pallas_call(
        body,
        out_shape=jax.ShapeDtypeStruct((n_rows, STATE_DIM), jnp.float32),
        in_specs=[
            pl.BlockSpec(memory_space=pltpu.SMEM),
            pl.BlockSpec(memory_space=pl.ANY),
        ],
        out_specs=pl.BlockSpec(memory_space=pl.ANY),
        scratch_shapes=[
            pltpu.VMEM((n_rows, STATE_DIM), jnp.float32),
            pltpu.SemaphoreType.DMA,
            pltpu.SemaphoreType.DMA,
        ],
    )


_sc_gather = _make_sc_gather(_SC_ROWS)
_tc_gather = _make_tc_gather(BATCH - _SC_ROWS) if _SC_ROWS < BATCH else None


def kernel(memory, last_update, n_id):
    nid = n_id.astype(jnp.int32)
    if _SC_ROWS == 0:
        (lu_out,) = _sc_gather(memory, last_update, nid)
        mem_out = _tc_gather(nid, memory)
        return (mem_out, lu_out)
    if _SC_ROWS == BATCH:
        lu_out, mem_out = _sc_gather(memory, last_update, nid)
        return (mem_out, lu_out)
    b_all = BATCH // _NW
    b_w = _SC_ROWS // _NW
    nid2 = nid.reshape(_NW, b_all)
    tc_ids = nid2[:, b_w:].reshape(-1)
    lu_out, sc_rows = _sc_gather(memory, last_update, nid)
    tc_rows = _tc_gather(tc_ids, memory)
    mem_out = jnp.concatenate(
        [sc_rows.reshape(_NW, b_w, STATE_DIM),
         tc_rows.reshape(_NW, b_all - b_w, STATE_DIM)], axis=1
    ).reshape(BATCH, STATE_DIM)
    return (mem_out, lu_out)


# speedup vs baseline: 1.0009x; 1.0009x over previous
import functools

import jax
import jax.numpy as jnp
from jax import lax
from jax.experimental import pallas as pl
from jax.experimental.pallas import tpu as pltpu
from jax.experimental.pallas import tpu_sc as plsc

NUM_NODES = 1000000
STATE_DIM = 64
BATCH = 16384

_info = plsc.get_sparse_core_info()
_NC, _NS, _L = _info.num_cores, _info.num_subcores, _info.num_lanes
_NW = _NC * _NS
_SC_ROWS = 8704
_K = 16
_NSEM = 16


def _make_sc_gather(sc_rows):
    mesh = plsc.VectorSubcoreMesh(core_axis_name="c", subcore_axis_name="s")
    b_w = sc_rows // _NW
    ng = b_w // _K if b_w else 0
    out_types = [jax.ShapeDtypeStruct((BATCH,), jnp.int32)]
    scratch = [
        pltpu.VMEM((BATCH // _NW,), jnp.int32),
        pltpu.VMEM((BATCH // _NW,), jnp.int32),
        pltpu.SemaphoreType.DMA,
    ]
    if sc_rows:
        out_types.append(
            jax.ShapeDtypeStruct((sc_rows, STATE_DIM), jnp.float32))
        scratch += [
            pltpu.VMEM((b_w, STATE_DIM), jnp.float32),
            pltpu.SemaphoreType.DMA,
            pltpu.SemaphoreType.DMA,
        ]

    @functools.partial(
        pl.kernel,
        mesh=mesh,
        out_type=tuple(out_types),
        scratch_types=scratch,
        compiler_params=pltpu.CompilerParams(needs_layout_passes=False),
        cost_estimate=pl.CostEstimate(
            flops=0, bytes_accessed=16 * BATCH * STATE_DIM * 4,
            transcendentals=0),
    )
    def k(mem_hbm, lu_hbm, nid_hbm, lu_out, *rest):
        wid = lax.axis_index("s") * _NC + lax.axis_index("c")
        lu_b = BATCH // _NW
        lu_base = wid * lu_b
        if sc_rows:
            mem_out, idx_v, lu_v, s_lu, rows_v, s_a, s_b = rest
        else:
            idx_v, lu_v, s_lu = rest
        pltpu.sync_copy(nid_hbm.at[pl.ds(lu_base, lu_b)], idx_v)
        lu_cp = pltpu.async_copy(lu_hbm.at[idx_v], lu_v, s_lu)

        if sc_rows:
            base = wid * b_w

            def body(g, _):
                vec = idx_v[pl.ds(g * _K, _K)]
                for b in range(_K):
                    v = vec[b]
                    pltpu.async_copy(
                        mem_hbm.at[v], rows_v.at[g * _K + b], s_a)
                return _

            lax.fori_loop(0, ng, body, 0)
            pltpu.make_async_copy(
                mem_hbm.at[pl.ds(0, b_w)], rows_v, s_a).wait()
            pltpu.sync_copy(rows_v, mem_out.at[pl.ds(base, b_w)])

        lu_cp.wait()
        pltpu.sync_copy(lu_v, lu_out.at[pl.ds(lu_base, lu_b)])

    return k


_TC_UNROLL = 8


def _make_tc_gather(n_rows):
    def body(nid_smem, mem_any, out_any, rows_v, sem, sem_out):
        def loop(jo, carry):
            for b in range(_TC_UNROLL):
                j = jo * _TC_UNROLL + b
                v = nid_smem[j]
                pltpu.make_async_copy(
                    mem_any.at[pl.ds(v, 1)], rows_v.at[pl.ds(j, 1)],
                    sem).start()
            return carry

        lax.fori_loop(0, n_rows // _TC_UNROLL, loop, 0)
        pltpu.make_async_copy(
            mem_any.at[pl.ds(0, n_rows)], rows_v, sem).wait()
        out_cp = pltpu.make_async_copy(rows_v, out_any, sem_out)
        out_cp.start()
        out_cp.wait()

    return pl.pallas_call(
        body,
        out_shape=jax.ShapeDtypeStruct((n_rows, STATE_DIM), jnp.float32),
        in_specs=[
            pl.BlockSpec(memory_space=pltpu.SMEM),
            pl.BlockSpec(memory_space=pl.ANY),
        ],
        out_specs=pl.BlockSpec(memory_space=pl.ANY),
        scratch_shapes=[
            pltpu.VMEM((n_rows, STATE_DIM), jnp.float32),
            pltpu.SemaphoreType.DMA,
            pltpu.SemaphoreType.DMA,
        ],
    )


_sc_gather = _make_sc_gather(_SC_ROWS)
_tc_gather = _make_tc_gather(BATCH - _SC_ROWS) if _SC_ROWS < BATCH else None


def kernel(memory, last_update, n_id):
    nid = n_id.astype(jnp.int32)
    if _SC_ROWS == 0:
        (lu_out,) = _sc_gather(memory, last_update, nid)
        mem_out = _tc_gather(nid, memory)
        return (mem_out, lu_out)
    if _SC_ROWS == BATCH:
        lu_out, mem_out = _sc_gather(memory, last_update, nid)
        return (mem_out, lu_out)
    b_all = BATCH // _NW
    b_w = _SC_ROWS // _NW
    nid2 = nid.reshape(_NW, b_all)
    tc_ids = nid2[:, b_w:].reshape(-1)
    lu_out, sc_rows = _sc_gather(memory, last_update, nid)
    tc_rows = _tc_gather(tc_ids, memory)
    mem_out = jnp.concatenate(
        [sc_rows.reshape(_NW, b_w, STATE_DIM),
         tc_rows.reshape(_NW, b_all - b_w, STATE_DIM)], axis=1
    ).reshape(BATCH, STATE_DIM)
    return (mem_out, lu_out)


# speedup vs baseline: 1.0953x; 1.0943x over previous
import functools

import jax
import jax.numpy as jnp
from jax import lax
from jax.experimental import pallas as pl
from jax.experimental.pallas import tpu as pltpu
from jax.experimental.pallas import tpu_sc as plsc

NUM_NODES = 1000000
STATE_DIM = 64
BATCH = 16384

_info = plsc.get_sparse_core_info()
_NC, _NS, _L = _info.num_cores, _info.num_subcores, _info.num_lanes
_NW = _NC * _NS
_B_PER_W = BATCH // _NW
_K = 16


def _make_gather():
    mesh = plsc.VectorSubcoreMesh(core_axis_name="c", subcore_axis_name="s")

    @functools.partial(
        pl.kernel,
        mesh=mesh,
        out_type=(
            jax.ShapeDtypeStruct((BATCH, STATE_DIM), jnp.float32),
            jax.ShapeDtypeStruct((BATCH,), jnp.int32),
        ),
        scratch_types=[
            pltpu.VMEM((_B_PER_W,), jnp.int32),
            pltpu.VMEM((_B_PER_W, STATE_DIM), jnp.float32),
            pltpu.VMEM((_B_PER_W,), jnp.int32),
            pltpu.SemaphoreType.DMA,
            pltpu.SemaphoreType.DMA,
        ],
    )
    def k(mem_hbm, lu_hbm, nid_hbm, mem_out, lu_out,
          idx_v, rows_v, lu_v, s_rows, s_lu):
        wid = lax.axis_index("s") * _NC + lax.axis_index("c")
        base = wid * _B_PER_W
        pltpu.sync_copy(nid_hbm.at[pl.ds(base, _B_PER_W)], idx_v)
        lu_cp = pltpu.async_copy(lu_hbm.at[idx_v], lu_v, s_lu)

        def body(g, carry):
            vec = idx_v[pl.ds(g * _K, _K)]
            for b in range(_K):
                v = vec[b]
                pltpu.async_copy(
                    mem_hbm.at[v], rows_v.at[g * _K + b], s_rows)
            return carry

        lax.fori_loop(0, _B_PER_W // _K, body, 0)
        pltpu.make_async_copy(
            mem_hbm.at[pl.ds(0, _B_PER_W)], rows_v, s_rows).wait()
        pltpu.sync_copy(rows_v, mem_out.at[pl.ds(base, _B_PER_W)])
        lu_cp.wait()
        pltpu.sync_copy(lu_v, lu_out.at[pl.ds(base, _B_PER_W)])

    return k


_gather = _make_gather()


def kernel(memory, last_update, n_id):
    return _gather(memory, last_update, n_id.astype(jnp.int32))
